# baseline (speedup 1.0000x reference)
import functools

import jax
import jax.numpy as jnp
from jax import lax
from jax.experimental import pallas as pl
from jax.experimental.pallas import tpu as pltpu
from jax.experimental.pallas import tpu_sc as plsc

NC = 2
NS = 16
NW = NC * NS
CH = 80
NP = 10240
NSL = 5


def _table_body(h_ref, x128_ref, wa_ref, wb_ref, trow_ref, tcol_ref):
    h = h_ref[...]
    x128 = x128_ref[...]
    H = h.shape[1]
    trow_ref[:, :H] = jnp.dot(h, wb_ref[...], preferred_element_type=jnp.float32)
    trow_ref[:, H:] = -x128
    tcol_ref[:, :H] = jnp.dot(h, wa_ref[...], preferred_element_type=jnp.float32)
    tcol_ref[:, H:] = x128


def _edge_body(gs_ref, gx_ref, ea_ref, col_ref, wd_ref, wde_ref, b1_ref,
               w2_ref, b2_ref, wg_ref, bg_ref, wc1_ref, bc1_ref,
               wc2_ref, bc2_ref, em_ref, ec_ref):
    H = w2_ref.shape[0]
    B = gs_ref.shape[0]
    s = gs_ref[...]
    cd16 = gx_ref[...]
    dist = jnp.sqrt(jnp.sum(cd16 * cd16, axis=1, keepdims=True))
    pre = (s + dist * wd_ref[...]
           + jnp.dot(ea_ref[...], wde_ref[...], preferred_element_type=jnp.float32)
           + b1_ref[...])
    t1 = pre * jax.nn.sigmoid(pre)
    bf16 = jnp.bfloat16
    t2 = jnp.dot(t1.astype(bf16), w2_ref[...].astype(bf16),
                 preferred_element_type=jnp.float32) + b2_ref[...]
    t2 = t2 * jax.nn.sigmoid(t2)
    gate = jax.nn.sigmoid(
        jnp.sum(t2 * wg_ref[...], axis=1, keepdims=True) + bg_ref[...])
    m = t2 * gate
    c1 = jnp.dot(m.astype(bf16), wc1_ref[...].astype(bf16),
                 preferred_element_type=jnp.float32) + bc1_ref[...]
    c1 = c1 * jax.nn.sigmoid(c1)
    cw = jnp.sum(c1 * wc2_ref[...], axis=1, keepdims=True) + bc2_ref[...]
    em_ref[...] = m
    cdw = cd16 * cw
    tiled = jnp.reshape(
        jnp.broadcast_to(jnp.reshape(cdw, (B, 1, 16)), (B, 8, 16)), (B, H))
    grp = lax.broadcasted_iota(jnp.int32, (B, H), 1) // 16
    ec_ref[...] = jnp.where((col_ref[...] % 8) == grp, tiled, 0.0)


def _node_body(*refs):
    h_ref, x16p_ref = refs[0], refs[1]
    mparts = refs[2:2 + NSL]
    cparts = refs[2 + NSL:2 + 2 * NSL]
    wn1a_ref, wn1b_ref, bn1_ref, wn2_ref, bn2_ref, hnew_ref, xnewp_ref = \
        refs[2 + 2 * NSL:]
    h = h_ref[...]
    magg = mparts[0][...]
    for p in mparts[1:]:
        magg = magg + p[...]
    csum = cparts[0][...]
    for p in cparts[1:]:
        csum = csum + p[...]
    u = (jnp.dot(h, wn1a_ref[...], preferred_element_type=jnp.float32)
         + jnp.dot(magg, wn1b_ref[...], preferred_element_type=jnp.float32)
         + bn1_ref[...])
    u = u * jax.nn.sigmoid(u)
    delta = jnp.dot(u, wn2_ref[...], preferred_element_type=jnp.float32) + bn2_ref[...]
    hnew_ref[...] = h + delta
    xnewp_ref[...] = x16p_ref[...] + csum


def _make_gather(E, DT):
    epw = E // NW
    nch = epw // CH
    H = DT // 2
    mesh = plsc.VectorSubcoreMesh(
        core_axis_name="c", subcore_axis_name="s", num_cores=NC, num_subcores=NS)

    @functools.partial(
        pl.kernel,
        out_type=[jax.ShapeDtypeStruct((E, H), jnp.float32),
                  jax.ShapeDtypeStruct((E, 16), jnp.float32)],
        mesh=mesh,
        scratch_types=[pltpu.VMEM((2, CH), jnp.int32),
                       pltpu.VMEM((2, CH), jnp.int32),
                       pltpu.VMEM((2, CH, DT), jnp.float32),
                       pltpu.VMEM((2, CH, DT), jnp.float32),
                       pltpu.VMEM((2, CH, 16), jnp.float32),
                       pltpu.SemaphoreType.DMA,
                       pltpu.SemaphoreType.DMA],
    )
    def gather_k(trow_hbm, tcol_hbm, ridx_hbm, cidx_hbm, gs_hbm, gx_hbm,
                 idxr_v, idxc_v, bufr_v, bufc_v, cd_v, sem0, sem1):
        c = lax.axis_index("c")
        s = lax.axis_index("s")
        wid = s * NC + c
        base = wid * epw
        sems = (sem0, sem1)

        def load_start(jj, slot):
            off = base + jj * CH
            pltpu.sync_copy(ridx_hbm.at[pl.ds(off, CH)], idxr_v.at[slot])
            pltpu.sync_copy(cidx_hbm.at[pl.ds(off, CH)], idxc_v.at[slot])
            pltpu.async_copy(trow_hbm.at[idxr_v.at[slot]], bufr_v.at[slot],
                             sems[slot])
            pltpu.async_copy(tcol_hbm.at[idxc_v.at[slot]], bufc_v.at[slot],
                             sems[slot])

        load_start(0, 0)

        @pl.loop(0, nch)
        def _chunk(j):
            for slot in (0, 1):
                @pl.when(j % 2 == slot)
                def _():
                    @pl.when(j + 1 < nch)
                    def _():
                        load_start(j + 1, 1 - slot)
                    pltpu.make_async_copy(
                        trow_hbm.at[idxr_v.at[slot]], bufr_v.at[slot],
                        sems[slot]).wait()
                    pltpu.make_async_copy(
                        tcol_hbm.at[idxc_v.at[slot]], bufc_v.at[slot],
                        sems[slot]).wait()

                    @pl.loop(0, CH)
                    def _edge(e):
                        for grp in range(8):
                            lsl = pl.ds(grp * 16, 16)
                            bufr_v[slot, e, lsl] = (bufr_v[slot, e, lsl]
                                                    + bufc_v[slot, e, lsl])
                        xsl = pl.ds(H, 16)
                        cd_v[slot, e, :] = (bufr_v[slot, e, xsl]
                                            + bufc_v[slot, e, xsl])

                    off = base + j * CH
                    pltpu.sync_copy(bufr_v.at[slot, :, pl.ds(0, H)],
                                    gs_hbm.at[pl.ds(off, CH)])
                    pltpu.sync_copy(cd_v.at[slot], gx_hbm.at[pl.ds(off, CH)])

    return gather_k


def _make_scatter(E, H):
    CHS = 80
    eps = E // NS
    nch = eps // CHS
    rpt = NP // NS
    npc = NP // 8
    cpt = npc // NS
    mesh = plsc.VectorSubcoreMesh(
        core_axis_name="c", subcore_axis_name="s", num_cores=NC, num_subcores=NS)

    @functools.partial(
        pl.kernel,
        out_type=[jax.ShapeDtypeStruct((NP, H), jnp.float32),
                  jax.ShapeDtypeStruct((npc, H), jnp.float32)],
        mesh=mesh,
        scratch_types=[pltpu.VMEM_SHARED((NP, H), jnp.float32),
                       pltpu.VMEM_SHARED((npc, H), jnp.float32),
                       pltpu.VMEM((2, CHS), jnp.int32),
                       pltpu.VMEM((2, CHS), jnp.int32),
                       pltpu.VMEM((2, CHS, H), jnp.float32),
                       pltpu.SemaphoreType.DMA,
                       pltpu.SemaphoreType.DMA],
    )
    def scatter_k(em_hbm, ec_hbm, cidx_hbm, zeros_hbm, outm_hbm, outc_hbm,
                  accm_sh, accc_sh, idx_v, idx2_v, dbuf_v, sem0, sem1):
        c = lax.axis_index("c")
        s = lax.axis_index("s")
        base = s * eps
        sems = (sem0, sem1)

        @pl.when(c == 0)
        def _():
            pltpu.sync_copy(zeros_hbm, accm_sh.at[pl.ds(s * rpt, rpt)])

        @pl.when(c == 1)
        def _():
            pltpu.sync_copy(zeros_hbm.at[pl.ds(0, cpt)],
                            accc_sh.at[pl.ds(s * cpt, cpt)])

        plsc.subcore_barrier()

        def load_start(jj, slot):
            off = base + jj * CHS
            pltpu.sync_copy(cidx_hbm.at[pl.ds(off, CHS)], idx_v.at[slot])

            @pl.when(c == 0)
            def _():
                pltpu.async_copy(em_hbm.at[pl.ds(off, CHS)], dbuf_v.at[slot],
                                 sems[slot])

            @pl.when(c == 1)
            def _():
                pltpu.async_copy(ec_hbm.at[pl.ds(off, CHS)], dbuf_v.at[slot],
                                 sems[slot])
                for q in range(CHS // 16):
                    sl = pl.ds(q * 16, 16)
                    idx2_v[slot, sl] = lax.shift_right_logical(
                        idx_v[slot, sl], 3)

        load_start(0, 0)

        @pl.loop(0, nch)
        def _chunk(j):
            for slot in (0, 1):
                @pl.when(j % 2 == slot)
                def _():
                    @pl.when(j + 1 < nch)
                    def _():
                        load_start(j + 1, 1 - slot)
                    off = base + j * CHS
                    pltpu.make_async_copy(
                        em_hbm.at[pl.ds(off, CHS)], dbuf_v.at[slot],
                        sems[slot]).wait()

                    @pl.when(c == 0)
                    def _():
                        pltpu.sync_copy(dbuf_v.at[slot],
                                        accm_sh.at[idx_v.at[slot]], add=True)

                    @pl.when(c == 1)
                    def _():
                        pltpu.sync_copy(dbuf_v.at[slot],
                                        accc_sh.at[idx2_v.at[slot]], add=True)

        plsc.subcore_barrier()

        @pl.when(c == 0)
        def _():
            pltpu.sync_copy(accm_sh.at[pl.ds(s * rpt, rpt)],
                            outm_hbm.at[pl.ds(s * rpt, rpt)])

        @pl.when(c == 1)
        def _():
            pltpu.sync_copy(accc_sh.at[pl.ds(s * cpt, cpt)],
                            outc_hbm.at[pl.ds(s * cpt, cpt)])

    return scatter_k


def kernel(h, x, edge_index, edge_attr, W_e1, b_e1, W_e2, b_e2, W_g, b_g,
           W_n1, b_n1, W_n2, b_n2, W_c1, b_c1, W_c2, b_c2):
    N, H = h.shape
    E = edge_index.shape[1]
    DE = edge_attr.shape[1]
    DT = 2 * H
    f32 = jnp.float32
    ESL = E // NSL

    row = edge_index[0]
    col = edge_index[1]
    col2d = col.reshape(E, 1)
    hp = jnp.pad(h, ((0, NP - N), (0, 0)))
    x16p = jnp.pad(x, ((0, NP - N), (0, 16 - x.shape[1]))).reshape(NP // 8, 8 * 16)
    x128 = jnp.pad(x, ((0, NP - N), (0, H - x.shape[1])))

    BN = 1024
    gn = NP // BN
    BC = BN // 8
    table = pl.pallas_call(
        _table_body,
        grid=(gn,),
        in_specs=[
            pl.BlockSpec((BN, H), lambda i: (i, 0)),
            pl.BlockSpec((BN, H), lambda i: (i, 0)),
            pl.BlockSpec((H, H), lambda i: (0, 0)),
            pl.BlockSpec((H, H), lambda i: (0, 0)),
        ],
        out_specs=[pl.BlockSpec((BN, DT), lambda i: (i, 0)),
                   pl.BlockSpec((BN, DT), lambda i: (i, 0))],
        out_shape=[jax.ShapeDtypeStruct((NP, DT), f32),
                   jax.ShapeDtypeStruct((NP, DT), f32)],
    )
    trow, tcol = table(hp, x128, W_e1[:H], W_e1[H:2 * H])

    gather = _make_gather(ESL, DT)
    scatter = _make_scatter(ESL, H)

    BE = 1280
    ge = ESL // BE
    edge_mlp = pl.pallas_call(
        _edge_body,
        grid=(ge,),
        in_specs=[
            pl.BlockSpec((BE, H), lambda i: (i, 0)),
            pl.BlockSpec((BE, 16), lambda i: (i, 0)),
            pl.BlockSpec((BE, DE), lambda i: (i, 0)),
            pl.BlockSpec((BE, 1), lambda i: (i, 0)),
            pl.BlockSpec((1, H), lambda i: (0, 0)),
            pl.BlockSpec((DE, H), lambda i: (0, 0)),
            pl.BlockSpec((1, H), lambda i: (0, 0)),
            pl.BlockSpec((H, H), lambda i: (0, 0)),
            pl.BlockSpec((1, H), lambda i: (0, 0)),
            pl.BlockSpec((1, H), lambda i: (0, 0)),
            pl.BlockSpec((1, 1), lambda i: (0, 0)),
            pl.BlockSpec((H, H), lambda i: (0, 0)),
            pl.BlockSpec((1, H), lambda i: (0, 0)),
            pl.BlockSpec((1, H), lambda i: (0, 0)),
            pl.BlockSpec((1, 1), lambda i: (0, 0)),
        ],
        out_specs=[pl.BlockSpec((BE, H), lambda i: (i, 0)),
                   pl.BlockSpec((BE, H), lambda i: (i, 0))],
        out_shape=[jax.ShapeDtypeStruct((ESL, H), f32),
                   jax.ShapeDtypeStruct((ESL, H), f32)],
    )

    zeros = jnp.zeros((NP // NS, H), f32)
    gathered = []
    for sl in range(NSL):
        lo = sl * ESL
        row_sl = lax.slice_in_dim(row, lo, lo + ESL)
        col_sl = lax.slice_in_dim(col, lo, lo + ESL)
        gathered.append((gather(trow, tcol, row_sl, col_sl), col_sl))

    edged = []
    for sl in range(NSL):
        lo = sl * ESL
        (gs, gx), col_sl = gathered[sl]
        em, ec = edge_mlp(
            gs, gx,
            lax.slice_in_dim(edge_attr, lo, lo + ESL, axis=0),
            lax.slice_in_dim(col2d, lo, lo + ESL, axis=0),
            W_e1[2 * H:2 * H + 1], W_e1[2 * H + 1:], b_e1.reshape(1, H),
            W_e2, b_e2.reshape(1, H), W_g.reshape(1, H), b_g.reshape(1, 1),
            W_c1, b_c1.reshape(1, H), W_c2.reshape(1, H), b_c2.reshape(1, 1))
        edged.append((em, ec, col_sl))

    mparts = []
    cparts = []
    for em, ec, col_sl in edged:
        m0, c0 = scatter(em, ec, col_sl, zeros)
        mparts.append(m0)
        cparts.append(c0)

    node = pl.pallas_call(
        _node_body,
        grid=(gn,),
        in_specs=(
            [pl.BlockSpec((BN, H), lambda i: (i, 0)),
             pl.BlockSpec((BC, H), lambda i: (i, 0))]
            + [pl.BlockSpec((BN, H), lambda i: (i, 0))] * NSL
            + [pl.BlockSpec((BC, H), lambda i: (i, 0))] * NSL
            + [pl.BlockSpec((H, H), lambda i: (0, 0)),
               pl.BlockSpec((H, H), lambda i: (0, 0)),
               pl.BlockSpec((1, H), lambda i: (0, 0)),
               pl.BlockSpec((H, H), lambda i: (0, 0)),
               pl.BlockSpec((1, H), lambda i: (0, 0))]),
        out_specs=[pl.BlockSpec((BN, H), lambda i: (i, 0)),
                   pl.BlockSpec((BC, H), lambda i: (i, 0))],
        out_shape=[jax.ShapeDtypeStruct((NP, H), f32),
                   jax.ShapeDtypeStruct((NP // 8, 8 * 16), f32)],
    )
    h_new, x_newp = node(hp, x16p, *mparts, *cparts,
                         W_n1[:H], W_n1[H:], b_n1.reshape(1, H),
                         W_n2, b_n2.reshape(1, H))
    x_new16 = x_newp.reshape(NP, 16)
    return (h_new[:N], x_new16[:N, :x.shape[1]])


# speedup vs baseline: 1.0279x; 1.0279x over previous
import functools

import jax
import jax.numpy as jnp
from jax import lax
from jax.experimental import pallas as pl
from jax.experimental.pallas import tpu as pltpu
from jax.experimental.pallas import tpu_sc as plsc

NC = 2
NS = 16
NW = NC * NS
CH = 80
NP = 10240
NSL = 5


def _table_body(h_ref, x128_ref, wa_ref, wb_ref, trow_ref, tcol_ref):
    h = h_ref[...]
    x128 = x128_ref[...]
    H = h.shape[1]
    trow_ref[:, :H] = jnp.dot(h, wb_ref[...], preferred_element_type=jnp.float32)
    trow_ref[:, H:] = -x128
    tcol_ref[:, :H] = jnp.dot(h, wa_ref[...], preferred_element_type=jnp.float32)
    tcol_ref[:, H:] = x128


def _edge_body(gs_ref, gx_ref, ea_ref, col_ref, wd_ref, wde_ref, b1_ref,
               w2_ref, b2_ref, wg_ref, bg_ref, wc1_ref, bc1_ref,
               wc2_ref, bc2_ref, em_ref, ec_ref):
    H = w2_ref.shape[0]
    B = gs_ref.shape[0]
    s = gs_ref[...]
    cd16 = gx_ref[...]
    dist = jnp.sqrt(jnp.sum(cd16 * cd16, axis=1, keepdims=True))
    pre = (s + dist * wd_ref[...]
           + jnp.dot(ea_ref[...], wde_ref[...], preferred_element_type=jnp.float32)
           + b1_ref[...])
    t1 = pre * jax.nn.sigmoid(pre)
    t2 = jnp.dot(t1, w2_ref[...], preferred_element_type=jnp.float32) + b2_ref[...]
    t2 = t2 * jax.nn.sigmoid(t2)
    gate = jax.nn.sigmoid(
        jnp.sum(t2 * wg_ref[...], axis=1, keepdims=True) + bg_ref[...])
    m = t2 * gate
    c1 = jnp.dot(m, wc1_ref[...], preferred_element_type=jnp.float32) + bc1_ref[...]
    c1 = c1 * jax.nn.sigmoid(c1)
    cw = jnp.sum(c1 * wc2_ref[...], axis=1, keepdims=True) + bc2_ref[...]
    em_ref[...] = m
    cdw = cd16 * cw
    tiled = jnp.reshape(
        jnp.broadcast_to(jnp.reshape(cdw, (B, 1, 16)), (B, 8, 16)), (B, H))
    grp = lax.broadcasted_iota(jnp.int32, (B, H), 1) // 16
    ec_ref[...] = jnp.where((col_ref[...] % 8) == grp, tiled, 0.0)


def _node_body(*refs):
    h_ref, x16p_ref = refs[0], refs[1]
    mparts = refs[2:2 + NSL]
    cparts = refs[2 + NSL:2 + 2 * NSL]
    wn1a_ref, wn1b_ref, bn1_ref, wn2_ref, bn2_ref, hnew_ref, xnewp_ref = \
        refs[2 + 2 * NSL:]
    h = h_ref[...]
    magg = mparts[0][...]
    for p in mparts[1:]:
        magg = magg + p[...]
    csum = cparts[0][...]
    for p in cparts[1:]:
        csum = csum + p[...]
    u = (jnp.dot(h, wn1a_ref[...], preferred_element_type=jnp.float32)
         + jnp.dot(magg, wn1b_ref[...], preferred_element_type=jnp.float32)
         + bn1_ref[...])
    u = u * jax.nn.sigmoid(u)
    delta = jnp.dot(u, wn2_ref[...], preferred_element_type=jnp.float32) + bn2_ref[...]
    hnew_ref[...] = h + delta
    xnewp_ref[...] = x16p_ref[...] + csum


def _make_gather(E, DT):
    epw = E // NW
    nch = epw // CH
    H = DT // 2
    mesh = plsc.VectorSubcoreMesh(
        core_axis_name="c", subcore_axis_name="s", num_cores=NC, num_subcores=NS)

    @functools.partial(
        pl.kernel,
        out_type=[jax.ShapeDtypeStruct((E, H), jnp.float32),
                  jax.ShapeDtypeStruct((E, 16), jnp.float32)],
        mesh=mesh,
        scratch_types=[pltpu.VMEM((2, CH), jnp.int32),
                       pltpu.VMEM((2, CH), jnp.int32),
                       pltpu.VMEM((2, CH, DT), jnp.float32),
                       pltpu.VMEM((2, CH, DT), jnp.float32),
                       pltpu.VMEM((2, CH, 16), jnp.float32),
                       pltpu.SemaphoreType.DMA,
                       pltpu.SemaphoreType.DMA],
    )
    def gather_k(trow_hbm, tcol_hbm, ridx_hbm, cidx_hbm, gs_hbm, gx_hbm,
                 idxr_v, idxc_v, bufr_v, bufc_v, cd_v, sem0, sem1):
        c = lax.axis_index("c")
        s = lax.axis_index("s")
        wid = s * NC + c
        base = wid * epw
        sems = (sem0, sem1)

        def load_start(jj, slot):
            off = base + jj * CH
            pltpu.sync_copy(ridx_hbm.at[pl.ds(off, CH)], idxr_v.at[slot])
            pltpu.sync_copy(cidx_hbm.at[pl.ds(off, CH)], idxc_v.at[slot])
            pltpu.async_copy(trow_hbm.at[idxr_v.at[slot]], bufr_v.at[slot],
                             sems[slot])
            pltpu.async_copy(tcol_hbm.at[idxc_v.at[slot]], bufc_v.at[slot],
                             sems[slot])

        load_start(0, 0)

        @pl.loop(0, nch)
        def _chunk(j):
            for slot in (0, 1):
                @pl.when(j % 2 == slot)
                def _():
                    @pl.when(j + 1 < nch)
                    def _():
                        load_start(j + 1, 1 - slot)
                    pltpu.make_async_copy(
                        trow_hbm.at[idxr_v.at[slot]], bufr_v.at[slot],
                        sems[slot]).wait()
                    pltpu.make_async_copy(
                        tcol_hbm.at[idxc_v.at[slot]], bufc_v.at[slot],
                        sems[slot]).wait()

                    @pl.loop(0, CH, unroll=8)
                    def _edge(e):
                        for grp in range(8):
                            lsl = pl.ds(grp * 16, 16)
                            bufr_v[slot, e, lsl] = (bufr_v[slot, e, lsl]
                                                    + bufc_v[slot, e, lsl])
                        xsl = pl.ds(H, 16)
                        cd_v[slot, e, :] = (bufr_v[slot, e, xsl]
                                            + bufc_v[slot, e, xsl])

                    off = base + j * CH
                    pltpu.sync_copy(bufr_v.at[slot, :, pl.ds(0, H)],
                                    gs_hbm.at[pl.ds(off, CH)])
                    pltpu.sync_copy(cd_v.at[slot], gx_hbm.at[pl.ds(off, CH)])

    return gather_k


def _make_scatter(E, H):
    CHS = 80
    eps = E // NS
    nch = eps // CHS
    rpt = NP // NS
    npc = NP // 8
    cpt = npc // NS
    mesh = plsc.VectorSubcoreMesh(
        core_axis_name="c", subcore_axis_name="s", num_cores=NC, num_subcores=NS)

    @functools.partial(
        pl.kernel,
        out_type=[jax.ShapeDtypeStruct((NP, H), jnp.float32),
                  jax.ShapeDtypeStruct((npc, H), jnp.float32)],
        mesh=mesh,
        scratch_types=[pltpu.VMEM_SHARED((NP, H), jnp.float32),
                       pltpu.VMEM_SHARED((npc, H), jnp.float32),
                       pltpu.VMEM((2, CHS), jnp.int32),
                       pltpu.VMEM((2, CHS), jnp.int32),
                       pltpu.VMEM((2, CHS, H), jnp.float32),
                       pltpu.SemaphoreType.DMA,
                       pltpu.SemaphoreType.DMA],
    )
    def scatter_k(em_hbm, ec_hbm, cidx_hbm, zeros_hbm, outm_hbm, outc_hbm,
                  accm_sh, accc_sh, idx_v, idx2_v, dbuf_v, sem0, sem1):
        c = lax.axis_index("c")
        s = lax.axis_index("s")
        base = s * eps
        sems = (sem0, sem1)

        @pl.when(c == 0)
        def _():
            pltpu.sync_copy(zeros_hbm, accm_sh.at[pl.ds(s * rpt, rpt)])

        @pl.when(c == 1)
        def _():
            pltpu.sync_copy(zeros_hbm.at[pl.ds(0, cpt)],
                            accc_sh.at[pl.ds(s * cpt, cpt)])

        plsc.subcore_barrier()

        def load_start(jj, slot):
            off = base + jj * CHS
            pltpu.sync_copy(cidx_hbm.at[pl.ds(off, CHS)], idx_v.at[slot])

            @pl.when(c == 0)
            def _():
                pltpu.async_copy(em_hbm.at[pl.ds(off, CHS)], dbuf_v.at[slot],
                                 sems[slot])

            @pl.when(c == 1)
            def _():
                pltpu.async_copy(ec_hbm.at[pl.ds(off, CHS)], dbuf_v.at[slot],
                                 sems[slot])
                for q in range(CHS // 16):
                    sl = pl.ds(q * 16, 16)
                    idx2_v[slot, sl] = lax.shift_right_logical(
                        idx_v[slot, sl], 3)

        load_start(0, 0)

        @pl.loop(0, nch)
        def _chunk(j):
            for slot in (0, 1):
                @pl.when(j % 2 == slot)
                def _():
                    @pl.when(j + 1 < nch)
                    def _():
                        load_start(j + 1, 1 - slot)
                    off = base + j * CHS
                    pltpu.make_async_copy(
                        em_hbm.at[pl.ds(off, CHS)], dbuf_v.at[slot],
                        sems[slot]).wait()

                    @pl.when(c == 0)
                    def _():
                        pltpu.sync_copy(dbuf_v.at[slot],
                                        accm_sh.at[idx_v.at[slot]], add=True)

                    @pl.when(c == 1)
                    def _():
                        pltpu.sync_copy(dbuf_v.at[slot],
                                        accc_sh.at[idx2_v.at[slot]], add=True)

        plsc.subcore_barrier()

        @pl.when(c == 0)
        def _():
            pltpu.sync_copy(accm_sh.at[pl.ds(s * rpt, rpt)],
                            outm_hbm.at[pl.ds(s * rpt, rpt)])

        @pl.when(c == 1)
        def _():
            pltpu.sync_copy(accc_sh.at[pl.ds(s * cpt, cpt)],
                            outc_hbm.at[pl.ds(s * cpt, cpt)])

    return scatter_k


def kernel(h, x, edge_index, edge_attr, W_e1, b_e1, W_e2, b_e2, W_g, b_g,
           W_n1, b_n1, W_n2, b_n2, W_c1, b_c1, W_c2, b_c2):
    N, H = h.shape
    E = edge_index.shape[1]
    DE = edge_attr.shape[1]
    DT = 2 * H
    f32 = jnp.float32
    ESL = E // NSL

    row = edge_index[0]
    col = edge_index[1]
    col2d = col.reshape(E, 1)
    hp = jnp.pad(h, ((0, NP - N), (0, 0)))
    x16p = jnp.pad(x, ((0, NP - N), (0, 16 - x.shape[1]))).reshape(NP // 8, 8 * 16)
    x128 = jnp.pad(x, ((0, NP - N), (0, H - x.shape[1])))

    BN = 1024
    gn = NP // BN
    BC = BN // 8
    table = pl.pallas_call(
        _table_body,
        grid=(gn,),
        in_specs=[
            pl.BlockSpec((BN, H), lambda i: (i, 0)),
            pl.BlockSpec((BN, H), lambda i: (i, 0)),
            pl.BlockSpec((H, H), lambda i: (0, 0)),
            pl.BlockSpec((H, H), lambda i: (0, 0)),
        ],
        out_specs=[pl.BlockSpec((BN, DT), lambda i: (i, 0)),
                   pl.BlockSpec((BN, DT), lambda i: (i, 0))],
        out_shape=[jax.ShapeDtypeStruct((NP, DT), f32),
                   jax.ShapeDtypeStruct((NP, DT), f32)],
    )
    trow, tcol = table(hp, x128, W_e1[:H], W_e1[H:2 * H])

    gather = _make_gather(ESL, DT)
    scatter = _make_scatter(ESL, H)

    BE = 1280
    ge = ESL // BE
    edge_mlp = pl.pallas_call(
        _edge_body,
        grid=(ge,),
        in_specs=[
            pl.BlockSpec((BE, H), lambda i: (i, 0)),
            pl.BlockSpec((BE, 16), lambda i: (i, 0)),
            pl.BlockSpec((BE, DE), lambda i: (i, 0)),
            pl.BlockSpec((BE, 1), lambda i: (i, 0)),
            pl.BlockSpec((1, H), lambda i: (0, 0)),
            pl.BlockSpec((DE, H), lambda i: (0, 0)),
            pl.BlockSpec((1, H), lambda i: (0, 0)),
            pl.BlockSpec((H, H), lambda i: (0, 0)),
            pl.BlockSpec((1, H), lambda i: (0, 0)),
            pl.BlockSpec((1, H), lambda i: (0, 0)),
            pl.BlockSpec((1, 1), lambda i: (0, 0)),
            pl.BlockSpec((H, H), lambda i: (0, 0)),
            pl.BlockSpec((1, H), lambda i: (0, 0)),
            pl.BlockSpec((1, H), lambda i: (0, 0)),
            pl.BlockSpec((1, 1), lambda i: (0, 0)),
        ],
        out_specs=[pl.BlockSpec((BE, H), lambda i: (i, 0)),
                   pl.BlockSpec((BE, H), lambda i: (i, 0))],
        out_shape=[jax.ShapeDtypeStruct((ESL, H), f32),
                   jax.ShapeDtypeStruct((ESL, H), f32)],
    )

    zeros = jnp.zeros((NP // NS, H), f32)
    gathered = []
    for sl in range(NSL):
        lo = sl * ESL
        row_sl = lax.slice_in_dim(row, lo, lo + ESL)
        col_sl = lax.slice_in_dim(col, lo, lo + ESL)
        gathered.append((gather(trow, tcol, row_sl, col_sl), col_sl))

    edged = []
    for sl in range(NSL):
        lo = sl * ESL
        (gs, gx), col_sl = gathered[sl]
        em, ec = edge_mlp(
            gs, gx,
            lax.slice_in_dim(edge_attr, lo, lo + ESL, axis=0),
            lax.slice_in_dim(col2d, lo, lo + ESL, axis=0),
            W_e1[2 * H:2 * H + 1], W_e1[2 * H + 1:], b_e1.reshape(1, H),
            W_e2, b_e2.reshape(1, H), W_g.reshape(1, H), b_g.reshape(1, 1),
            W_c1, b_c1.reshape(1, H), W_c2.reshape(1, H), b_c2.reshape(1, 1))
        edged.append((em, ec, col_sl))

    mparts = []
    cparts = []
    for em, ec, col_sl in edged:
        m0, c0 = scatter(em, ec, col_sl, zeros)
        mparts.append(m0)
        cparts.append(c0)

    node = pl.pallas_call(
        _node_body,
        grid=(gn,),
        in_specs=(
            [pl.BlockSpec((BN, H), lambda i: (i, 0)),
             pl.BlockSpec((BC, H), lambda i: (i, 0))]
            + [pl.BlockSpec((BN, H), lambda i: (i, 0))] * NSL
            + [pl.BlockSpec((BC, H), lambda i: (i, 0))] * NSL
            + [pl.BlockSpec((H, H), lambda i: (0, 0)),
               pl.BlockSpec((H, H), lambda i: (0, 0)),
               pl.BlockSpec((1, H), lambda i: (0, 0)),
               pl.BlockSpec((H, H), lambda i: (0, 0)),
               pl.BlockSpec((1, H), lambda i: (0, 0))]),
        out_specs=[pl.BlockSpec((BN, H), lambda i: (i, 0)),
                   pl.BlockSpec((BC, H), lambda i: (i, 0))],
        out_shape=[jax.ShapeDtypeStruct((NP, H), f32),
                   jax.ShapeDtypeStruct((NP // 8, 8 * 16), f32)],
    )
    h_new, x_newp = node(hp, x16p, *mparts, *cparts,
                         W_n1[:H], W_n1[H:], b_n1.reshape(1, H),
                         W_n2, b_n2.reshape(1, H))
    x_new16 = x_newp.reshape(NP, 16)
    return (h_new[:N], x_new16[:N, :x.shape[1]])


# speedup vs baseline: 1.1797x; 1.1477x over previous
import functools

import jax
import jax.numpy as jnp
from jax import lax
from jax.experimental import pallas as pl
from jax.experimental.pallas import tpu as pltpu
from jax.experimental.pallas import tpu_sc as plsc

NC = 2
NS = 16
NW = NC * NS
CH = 80
NP = 10240
NSL = 5


def _table_body(h_ref, x128_ref, wa_ref, wb_ref, trow_ref, tcol_ref):
    h = h_ref[...]
    x128 = x128_ref[...]
    H = h.shape[1]
    trow_ref[:, :H] = jnp.dot(h, wb_ref[...], preferred_element_type=jnp.float32)
    trow_ref[:, H:] = -x128
    tcol_ref[:, :H] = jnp.dot(h, wa_ref[...], preferred_element_type=jnp.float32)
    tcol_ref[:, H:] = x128


def _edge_body(gs_ref, gx_ref, ea_ref, col_ref, wd_ref, wde_ref, b1_ref,
               w2_ref, b2_ref, wg_ref, bg_ref, wc1_ref, bc1_ref,
               wc2_ref, bc2_ref, em_ref, ec_ref):
    H = w2_ref.shape[0]
    B = gs_ref.shape[0]
    s = gs_ref[...]
    cd16 = gx_ref[...]
    dist = jnp.sqrt(jnp.sum(cd16 * cd16, axis=1, keepdims=True))
    pre = (s + dist * wd_ref[...]
           + jnp.dot(ea_ref[...], wde_ref[...], preferred_element_type=jnp.float32)
           + b1_ref[...])
    t1 = pre * jax.nn.sigmoid(pre)
    t2 = jnp.dot(t1, w2_ref[...], preferred_element_type=jnp.float32) + b2_ref[...]
    t2 = t2 * jax.nn.sigmoid(t2)
    gate = jax.nn.sigmoid(
        jnp.sum(t2 * wg_ref[...], axis=1, keepdims=True) + bg_ref[...])
    m = t2 * gate
    c1 = jnp.dot(m, wc1_ref[...], preferred_element_type=jnp.float32) + bc1_ref[...]
    c1 = c1 * jax.nn.sigmoid(c1)
    cw = jnp.sum(c1 * wc2_ref[...], axis=1, keepdims=True) + bc2_ref[...]
    em_ref[...] = m
    cdw = cd16 * cw
    tiled = jnp.reshape(
        jnp.broadcast_to(jnp.reshape(cdw, (B, 1, 16)), (B, 8, 16)), (B, H))
    grp = lax.broadcasted_iota(jnp.int32, (B, H), 1) // 16
    ec_ref[...] = jnp.where((col_ref[...] % 8) == grp, tiled, 0.0)


def _node_body(*refs):
    h_ref, x16p_ref = refs[0], refs[1]
    mparts = refs[2:2 + NSL]
    cparts = refs[2 + NSL:2 + 2 * NSL]
    wn1a_ref, wn1b_ref, bn1_ref, wn2_ref, bn2_ref, hnew_ref, xnewp_ref = \
        refs[2 + 2 * NSL:]
    h = h_ref[...]
    magg = mparts[0][...]
    for p in mparts[1:]:
        magg = magg + p[...]
    csum = cparts[0][...]
    for p in cparts[1:]:
        csum = csum + p[...]
    u = (jnp.dot(h, wn1a_ref[...], preferred_element_type=jnp.float32)
         + jnp.dot(magg, wn1b_ref[...], preferred_element_type=jnp.float32)
         + bn1_ref[...])
    u = u * jax.nn.sigmoid(u)
    delta = jnp.dot(u, wn2_ref[...], preferred_element_type=jnp.float32) + bn2_ref[...]
    hnew_ref[...] = h + delta
    xnewp_ref[...] = x16p_ref[...] + csum


def _make_gather(E, DT):
    epw = E // NW
    nch = epw // CH
    H = DT // 2
    mesh = plsc.VectorSubcoreMesh(
        core_axis_name="c", subcore_axis_name="s", num_cores=NC, num_subcores=NS)

    @functools.partial(
        pl.kernel,
        out_type=[jax.ShapeDtypeStruct((E, H), jnp.float32),
                  jax.ShapeDtypeStruct((E, 16), jnp.float32)],
        mesh=mesh,
        scratch_types=[pltpu.VMEM((2, CH), jnp.int32),
                       pltpu.VMEM((2, CH), jnp.int32),
                       pltpu.VMEM((2, CH, DT), jnp.float32),
                       pltpu.VMEM((2, CH, DT), jnp.float32),
                       pltpu.VMEM((2, CH, 16), jnp.float32),
                       pltpu.SemaphoreType.DMA,
                       pltpu.SemaphoreType.DMA],
    )
    def gather_k(trow_hbm, tcol_hbm, ridx_hbm, cidx_hbm, gs_hbm, gx_hbm,
                 idxr_v, idxc_v, bufr_v, bufc_v, cd_v, sem0, sem1):
        c = lax.axis_index("c")
        s = lax.axis_index("s")
        wid = s * NC + c
        base = wid * epw
        sems = (sem0, sem1)

        def load_start(jj, slot):
            off = base + jj * CH
            pltpu.sync_copy(ridx_hbm.at[pl.ds(off, CH)], idxr_v.at[slot])
            pltpu.sync_copy(cidx_hbm.at[pl.ds(off, CH)], idxc_v.at[slot])
            pltpu.async_copy(trow_hbm.at[idxr_v.at[slot]], bufr_v.at[slot],
                             sems[slot])
            pltpu.async_copy(tcol_hbm.at[idxc_v.at[slot]], bufc_v.at[slot],
                             sems[slot])

        load_start(0, 0)

        @pl.loop(0, nch)
        def _chunk(j):
            for slot in (0, 1):
                @pl.when(j % 2 == slot)
                def _():
                    @pl.when(j + 1 < nch)
                    def _():
                        load_start(j + 1, 1 - slot)
                    pltpu.make_async_copy(
                        trow_hbm.at[idxr_v.at[slot]], bufr_v.at[slot],
                        sems[slot]).wait()
                    pltpu.make_async_copy(
                        tcol_hbm.at[idxc_v.at[slot]], bufc_v.at[slot],
                        sems[slot]).wait()

                    @pl.loop(0, CH, unroll=8)
                    def _edge(e):
                        for grp in range(8):
                            lsl = pl.ds(grp * 16, 16)
                            bufr_v[slot, e, lsl] = (bufr_v[slot, e, lsl]
                                                    + bufc_v[slot, e, lsl])
                        xsl = pl.ds(H, 16)
                        cd_v[slot, e, :] = (bufr_v[slot, e, xsl]
                                            + bufc_v[slot, e, xsl])

                    off = base + j * CH
                    pltpu.sync_copy(bufr_v.at[slot, :, pl.ds(0, H)],
                                    gs_hbm.at[pl.ds(off, CH)])
                    pltpu.sync_copy(cd_v.at[slot], gx_hbm.at[pl.ds(off, CH)])

    return gather_k


def _make_scatter(E, H):
    CHS = 80
    eps = E // NS
    nch = eps // CHS
    rpt = NP // NS
    npc = NP // 8
    cpt = npc // NS
    mesh = plsc.VectorSubcoreMesh(
        core_axis_name="c", subcore_axis_name="s", num_cores=NC, num_subcores=NS)

    @functools.partial(
        pl.kernel,
        out_type=[jax.ShapeDtypeStruct((NP, H), jnp.float32),
                  jax.ShapeDtypeStruct((npc, H), jnp.float32)],
        mesh=mesh,
        scratch_types=[pltpu.VMEM_SHARED((NP, H), jnp.float32),
                       pltpu.VMEM_SHARED((npc, H), jnp.float32),
                       pltpu.VMEM((2, CHS), jnp.int32),
                       pltpu.VMEM((2, CHS), jnp.int32),
                       pltpu.VMEM((2, CHS, H), jnp.float32),
                       pltpu.SemaphoreType.DMA,
                       pltpu.SemaphoreType.DMA],
    )
    def scatter_k(em_hbm, ec_hbm, cidx_hbm, zeros_hbm, outm_hbm, outc_hbm,
                  accm_sh, accc_sh, idx_v, idx2_v, dbuf_v, sem0, sem1):
        c = lax.axis_index("c")
        s = lax.axis_index("s")
        base = s * eps
        sems = (sem0, sem1)

        @pl.when(c == 0)
        def _():
            pltpu.sync_copy(zeros_hbm, accm_sh.at[pl.ds(s * rpt, rpt)])

        @pl.when(c == 1)
        def _():
            pltpu.sync_copy(zeros_hbm.at[pl.ds(0, cpt)],
                            accc_sh.at[pl.ds(s * cpt, cpt)])

        plsc.subcore_barrier()

        def load_start(jj, slot):
            off = base + jj * CHS
            pltpu.sync_copy(cidx_hbm.at[pl.ds(off, CHS)], idx_v.at[slot])

            @pl.when(c == 0)
            def _():
                pltpu.async_copy(em_hbm.at[pl.ds(off, CHS)], dbuf_v.at[slot],
                                 sems[slot])

            @pl.when(c == 1)
            def _():
                pltpu.async_copy(ec_hbm.at[pl.ds(off, CHS)], dbuf_v.at[slot],
                                 sems[slot])
                for q in range(CHS // 16):
                    sl = pl.ds(q * 16, 16)
                    idx2_v[slot, sl] = lax.shift_right_logical(
                        idx_v[slot, sl], 3)

        load_start(0, 0)

        @pl.loop(0, nch)
        def _chunk(j):
            for slot in (0, 1):
                @pl.when(j % 2 == slot)
                def _():
                    @pl.when(j + 1 < nch)
                    def _():
                        load_start(j + 1, 1 - slot)
                    off = base + j * CHS
                    pltpu.make_async_copy(
                        em_hbm.at[pl.ds(off, CHS)], dbuf_v.at[slot],
                        sems[slot]).wait()

                    @pl.when(c == 0)
                    def _():
                        pltpu.sync_copy(dbuf_v.at[slot],
                                        accm_sh.at[idx_v.at[slot]], add=True)

                    @pl.when(c == 1)
                    def _():
                        pltpu.sync_copy(dbuf_v.at[slot],
                                        accc_sh.at[idx2_v.at[slot]], add=True)

        plsc.subcore_barrier()

        @pl.when(c == 0)
        def _():
            pltpu.sync_copy(accm_sh.at[pl.ds(s * rpt, rpt)],
                            outm_hbm.at[pl.ds(s * rpt, rpt)])

        @pl.when(c == 1)
        def _():
            pltpu.sync_copy(accc_sh.at[pl.ds(s * cpt, cpt)],
                            outc_hbm.at[pl.ds(s * cpt, cpt)])

    return scatter_k


def kernel(h, x, edge_index, edge_attr, W_e1, b_e1, W_e2, b_e2, W_g, b_g,
           W_n1, b_n1, W_n2, b_n2, W_c1, b_c1, W_c2, b_c2):
    N, H = h.shape
    E = edge_index.shape[1]
    DE = edge_attr.shape[1]
    DT = 2 * H
    f32 = jnp.float32
    ESL = E // NSL

    row = edge_index[0]
    col = edge_index[1]
    col2d = col.reshape(E, 1)
    hp = jnp.pad(h, ((0, NP - N), (0, 0)))
    x16p = jnp.pad(x, ((0, NP - N), (0, 16 - x.shape[1]))).reshape(NP // 8, 8 * 16)
    x128 = jnp.pad(x, ((0, NP - N), (0, H - x.shape[1])))

    BN = 1024
    gn = NP // BN
    BC = BN // 8
    table = pl.pallas_call(
        _table_body,
        grid=(gn,),
        in_specs=[
            pl.BlockSpec((BN, H), lambda i: (i, 0)),
            pl.BlockSpec((BN, H), lambda i: (i, 0)),
            pl.BlockSpec((H, H), lambda i: (0, 0)),
            pl.BlockSpec((H, H), lambda i: (0, 0)),
        ],
        out_specs=[pl.BlockSpec((BN, DT), lambda i: (i, 0)),
                   pl.BlockSpec((BN, DT), lambda i: (i, 0))],
        out_shape=[jax.ShapeDtypeStruct((NP, DT), f32),
                   jax.ShapeDtypeStruct((NP, DT), f32)],
    )
    trow, tcol = table(hp, x128, W_e1[:H], W_e1[H:2 * H])

    gather = _make_gather(ESL, DT)
    scatter = _make_scatter(ESL, H)

    BE = 2560
    ge = ESL // BE
    edge_mlp = pl.pallas_call(
        _edge_body,
        grid=(ge,),
        in_specs=[
            pl.BlockSpec((BE, H), lambda i: (i, 0)),
            pl.BlockSpec((BE, 16), lambda i: (i, 0)),
            pl.BlockSpec((BE, DE), lambda i: (i, 0)),
            pl.BlockSpec((BE, 1), lambda i: (i, 0)),
            pl.BlockSpec((1, H), lambda i: (0, 0)),
            pl.BlockSpec((DE, H), lambda i: (0, 0)),
            pl.BlockSpec((1, H), lambda i: (0, 0)),
            pl.BlockSpec((H, H), lambda i: (0, 0)),
            pl.BlockSpec((1, H), lambda i: (0, 0)),
            pl.BlockSpec((1, H), lambda i: (0, 0)),
            pl.BlockSpec((1, 1), lambda i: (0, 0)),
            pl.BlockSpec((H, H), lambda i: (0, 0)),
            pl.BlockSpec((1, H), lambda i: (0, 0)),
            pl.BlockSpec((1, H), lambda i: (0, 0)),
            pl.BlockSpec((1, 1), lambda i: (0, 0)),
        ],
        out_specs=[pl.BlockSpec((BE, H), lambda i: (i, 0)),
                   pl.BlockSpec((BE, H), lambda i: (i, 0))],
        out_shape=[jax.ShapeDtypeStruct((ESL, H), f32),
                   jax.ShapeDtypeStruct((ESL, H), f32)],
    )

    zeros = jnp.zeros((NP // NS, H), f32)
    gathered = []
    for sl in range(NSL):
        lo = sl * ESL
        row_sl = lax.slice_in_dim(row, lo, lo + ESL)
        col_sl = lax.slice_in_dim(col, lo, lo + ESL)
        gathered.append((gather(trow, tcol, row_sl, col_sl), col_sl))

    edged = []
    for sl in range(NSL):
        lo = sl * ESL
        (gs, gx), col_sl = gathered[sl]
        em, ec = edge_mlp(
            gs, gx,
            lax.slice_in_dim(edge_attr, lo, lo + ESL, axis=0),
            lax.slice_in_dim(col2d, lo, lo + ESL, axis=0),
            W_e1[2 * H:2 * H + 1], W_e1[2 * H + 1:], b_e1.reshape(1, H),
            W_e2, b_e2.reshape(1, H), W_g.reshape(1, H), b_g.reshape(1, 1),
            W_c1, b_c1.reshape(1, H), W_c2.reshape(1, H), b_c2.reshape(1, 1))
        edged.append((em, ec, col_sl))

    mparts = []
    cparts = []
    for em, ec, col_sl in edged:
        m0, c0 = scatter(em, ec, col_sl, zeros)
        mparts.append(m0)
        cparts.append(c0)

    node = pl.pallas_call(
        _node_body,
        grid=(gn,),
        in_specs=(
            [pl.BlockSpec((BN, H), lambda i: (i, 0)),
             pl.BlockSpec((BC, H), lambda i: (i, 0))]
            + [pl.BlockSpec((BN, H), lambda i: (i, 0))] * NSL
            + [pl.BlockSpec((BC, H), lambda i: (i, 0))] * NSL
            + [pl.BlockSpec((H, H), lambda i: (0, 0)),
               pl.BlockSpec((H, H), lambda i: (0, 0)),
               pl.BlockSpec((1, H), lambda i: (0, 0)),
               pl.BlockSpec((H, H), lambda i: (0, 0)),
               pl.BlockSpec((1, H), lambda i: (0, 0))]),
        out_specs=[pl.BlockSpec((BN, H), lambda i: (i, 0)),
                   pl.BlockSpec((BC, H), lambda i: (i, 0))],
        out_shape=[jax.ShapeDtypeStruct((NP, H), f32),
                   jax.ShapeDtypeStruct((NP // 8, 8 * 16), f32)],
    )
    h_new, x_newp = node(hp, x16p, *mparts, *cparts,
                         W_n1[:H], W_n1[H:], b_n1.reshape(1, H),
                         W_n2, b_n2.reshape(1, H))
    x_new16 = x_newp.reshape(NP, 16)
    return (h_new[:N], x_new16[:N, :x.shape[1]])
